# baseline (speedup 1.0000x reference)
import functools

import jax
import jax.numpy as jnp
from jax import lax
from jax.experimental import pallas as pl
from jax.experimental.pallas import tpu as pltpu
from jax.experimental.pallas import tpu_sc as plsc

N = 1048576
D = 32
OUT = 16
LANES = 16
NUM_CORES = 2
NUM_SUBCORES = 16
NW = NUM_CORES * NUM_SUBCORES
ROWS_W = N // NW
CHUNK = 1024
NCHUNK = ROWS_W // CHUNK
GROUPS = CHUNK // LANES

_mesh = plsc.VectorSubcoreMesh(core_axis_name="c", subcore_axis_name="s")


@functools.partial(
    pl.kernel,
    out_type=jax.ShapeDtypeStruct((N,), jnp.float32),
    mesh=_mesh,
    scratch_types=[
        pltpu.VMEM((CHUNK, D), jnp.float32),
        pltpu.VMEM((CHUNK, D), jnp.float32),
        pltpu.VMEM((ROWS_W,), jnp.float32),
        pltpu.VMEM((OUT * D,), jnp.float32),
        pltpu.VMEM((OUT,), jnp.float32),
        pltpu.VMEM((2 * D,), jnp.float32),
        pltpu.SemaphoreType.DMA,
        pltpu.SemaphoreType.DMA,
    ],
    compiler_params=pltpu.CompilerParams(
        needs_layout_passes=False, use_tc_tiling_on_sc=False),
)
def _sc_rowsum(x_hbm, w_hbm, b_hbm, out_hbm,
               buf0, buf1, obuf, wvm, bvm, wdup, sem0, sem1):
    wid = lax.axis_index("s") * NUM_CORES + lax.axis_index("c")
    row0 = wid * ROWS_W

    def in_copy(c, buf, sem):
        return pltpu.make_async_copy(
            x_hbm.at[pl.ds(row0 + c * CHUNK, CHUNK), :], buf, sem)

    in_copy(0, buf0, sem0).start()
    in_copy(1, buf1, sem1).start()

    pltpu.sync_copy(w_hbm, wvm)
    pltpu.sync_copy(b_hbm, bvm)
    bvec = bvm[...]
    bsum = bvec[0]
    for o in range(1, OUT):
        bsum = bsum + bvec[o]
    for h in range(D // LANES):
        acc = wvm[pl.ds(h * LANES, LANES)]
        for o in range(1, OUT):
            acc = acc + wvm[pl.ds(o * D + h * LANES, LANES)]
        wdup[pl.ds(h * LANES, LANES)] = acc
        wdup[pl.ds(D + h * LANES, LANES)] = acc
    wrot = [wdup[pl.ds(j, LANES)] for j in range(D)]

    lane_iota = lax.iota(jnp.int32, LANES)
    diag_col = [((lane_iota + j) & (D - 1)) for j in range(D)]
    bsum_vec = jnp.broadcast_to(bsum, (LANES,))

    def chunk_compute(c, buf):
        @plsc.parallel_loop(0, GROUPS, unroll=2)
        def gbody(g):
            rows = lane_iota + g * LANES
            accs = [None] * 4
            for j in range(D):
                col = plsc.load_gather(buf, [rows, diag_col[j]])
                t = col * wrot[j]
                k = j % 4
                accs[k] = t if accs[k] is None else accs[k] + t
            acc = (accs[0] + accs[1]) + (accs[2] + accs[3]) + bsum_vec
            obuf[pl.ds(c * CHUNK + g * LANES, LANES)] = acc

    def pair_body(i, carry):
        for par, (buf, sem) in enumerate(((buf0, sem0), (buf1, sem1))):
            c = i * 2 + par
            in_copy(c, buf, sem).wait()
            chunk_compute(c, buf)

            @pl.when(i < NCHUNK // 2 - 1)
            def _():
                in_copy(c + 2, buf, sem).start()
        return carry

    lax.fori_loop(0, NCHUNK // 2, pair_body, 0)

    pltpu.sync_copy(obuf, out_hbm.at[pl.ds(row0, ROWS_W)])


def kernel(x, W, b):
    return _sc_rowsum(x, W.reshape(-1), b)


# speedup vs baseline: 7.9531x; 7.9531x over previous
import functools

import jax
import jax.numpy as jnp
from jax import lax
from jax.experimental import pallas as pl
from jax.experimental.pallas import tpu as pltpu
from jax.experimental.pallas import tpu_sc as plsc

N = 1048576
D = 32
OUT = 16
LANES = 16
NUM_CORES = 2
NUM_SUBCORES = 16
NW = NUM_CORES * NUM_SUBCORES
ROWS_W = N // NW
CW = 1024
NCHUNK = ROWS_W // CW
GROUPS = CW // LANES

_mesh = plsc.VectorSubcoreMesh(core_axis_name="c", subcore_axis_name="s")


@functools.partial(
    pl.kernel,
    out_type=jax.ShapeDtypeStruct((N,), jnp.float32),
    mesh=_mesh,
    scratch_types=[
        pltpu.VMEM((D, CW), jnp.float32),
        pltpu.VMEM((D, CW), jnp.float32),
        pltpu.VMEM((ROWS_W,), jnp.float32),
        pltpu.VMEM((OUT * D,), jnp.float32),
        pltpu.VMEM((OUT,), jnp.float32),
        pltpu.SemaphoreType.DMA,
        pltpu.SemaphoreType.DMA,
    ],
    compiler_params=pltpu.CompilerParams(
        needs_layout_passes=False, use_tc_tiling_on_sc=True),
)
def _sc_rowsum(xt_hbm, w_hbm, b_hbm, out_hbm,
               buf0, buf1, obuf, wvm, bvm, sem0, sem1):
    wid = lax.axis_index("s") * NUM_CORES + lax.axis_index("c")
    row0 = wid * ROWS_W

    def in_copy(c, buf, sem):
        return pltpu.make_async_copy(
            xt_hbm.at[:, pl.ds(row0 + c * CW, CW)], buf, sem)

    in_copy(0, buf0, sem0).start()
    in_copy(1, buf1, sem1).start()

    pltpu.sync_copy(w_hbm, wvm)
    pltpu.sync_copy(b_hbm, bvm)
    bvec = bvm[...]
    bsum = bvec[0]
    for o in range(1, OUT):
        bsum = bsum + bvec[o]
    wj = []
    for h in range(D // LANES):
        acc = wvm[pl.ds(h * LANES, LANES)]
        for o in range(1, OUT):
            acc = acc + wvm[pl.ds(o * D + h * LANES, LANES)]
        wj.extend(acc[j] for j in range(LANES))
    wjb = [jnp.broadcast_to(w, (LANES,)) for w in wj]
    bsum_vec = jnp.broadcast_to(bsum, (LANES,))

    def chunk_compute(c, buf):
        @plsc.parallel_loop(0, GROUPS, unroll=2)
        def gbody(g):
            r = g * LANES
            accs = [None] * 4
            for j in range(D):
                t = buf[j, pl.ds(r, LANES)] * wjb[j]
                k = j % 4
                accs[k] = t if accs[k] is None else accs[k] + t
            acc = (accs[0] + accs[1]) + (accs[2] + accs[3]) + bsum_vec
            obuf[pl.ds(c * CW + r, LANES)] = acc

    def pair_body(i, carry):
        for par, (buf, sem) in enumerate(((buf0, sem0), (buf1, sem1))):
            c = i * 2 + par
            in_copy(c, buf, sem).wait()
            chunk_compute(c, buf)

            @pl.when(i < NCHUNK // 2 - 1)
            def _():
                in_copy(c + 2, buf, sem).start()
        return carry

    lax.fori_loop(0, NCHUNK // 2, pair_body, 0)

    pltpu.sync_copy(obuf, out_hbm.at[pl.ds(row0, ROWS_W)])


def kernel(x, W, b):
    return _sc_rowsum(x.T, W.reshape(-1), b)
